# Initial kernel scaffold; baseline (speedup 1.0000x reference)
import jax
import jax.numpy as jnp
from jax.experimental import pallas as pl


def kernel(x, edge_index, edge_attr, pos, batch, n1_w1, n1_w2, n1_b2, conv1_root, conv1_bias, pool1_w, n2_w1, n2_w2, n2_b2, conv2_root, conv2_bias, pool2_w, mlp1_W, mlp1_b, bn1_g, bn1_b, mlp2_W, mlp2_b, bn2_g, bn2_b, smx_W, smx_b):
    raise NotImplementedError("write your pallas kernel here")


# speedup vs baseline: 5.3866x; 5.3866x over previous
import functools

import jax
import jax.numpy as jnp
from jax import lax
from jax.experimental import pallas as pl
from jax.experimental.pallas import tpu as pltpu
from jax.experimental.pallas import tpu_sc as plsc

_B = 64
_R = 128
_E = 131072
_NCLUST = 7
_D1 = 32
_D2 = 32
_K1 = 64
_K2 = 32

_EROWS = _E // 128
_NW = 32
_ROWS_W = _EROWS // _NW
_AWORDS = _B * _R * _R
_TILE_W = _AWORDS // 16


def _adj_body(src_hbm, dst_hbm, attr_hbm, out_hbm, srcv, dstv, attrv, idxv,
              zv, abuf, sem):
    cid = lax.axis_index("c")
    sid = lax.axis_index("s")
    wid = cid * 16 + sid
    base = wid * _ROWS_W

    pltpu.sync_copy(src_hbm.at[pl.ds(base, _ROWS_W)], srcv)
    pltpu.sync_copy(dst_hbm.at[pl.ds(base, _ROWS_W)], dstv)
    pltpu.sync_copy(attr_hbm.at[pl.ds(base, _ROWS_W)], attrv)

    def zbody(i, carry):
        zv[pl.ds(i * 16, 16)] = jnp.zeros((16,), jnp.float32)
        return carry
    lax.fori_loop(0, 256, zbody, 0)
    zbase = sid * _TILE_W
    for k in range(_TILE_W // 4096):
        pltpu.sync_copy(zv, abuf.at[pl.ds(zbase + k * 4096, 4096)])

    def ibody(j, carry):
        for c in range(8):
            s = srcv[j, pl.ds(c * 16, 16)]
            d = dstv[j, pl.ds(c * 16, 16)]
            idxv[j, pl.ds(c * 16, 16)] = d * _R + (s & (_R - 1))
        return carry
    lax.fori_loop(0, _ROWS_W, ibody, 0)

    plsc.subcore_barrier()

    cps = [pltpu.async_copy(attrv.at[j], abuf.at[idxv.at[j]], sem, add=True)
           for j in range(_ROWS_W)]
    for cp in cps:
        cp.wait()

    plsc.subcore_barrier()

    pltpu.sync_copy(abuf.at[pl.ds(zbase, _TILE_W)],
                    out_hbm.at[cid, pl.ds(zbase, _TILE_W)])


@functools.partial(jax.jit, static_argnames=())
def _build_adj(src, dst, attr):
    mesh = plsc.VectorSubcoreMesh(core_axis_name="c", subcore_axis_name="s")
    run = pl.kernel(
        _adj_body,
        out_type=jax.ShapeDtypeStruct((2, _AWORDS), jnp.float32),
        mesh=mesh,
        scratch_types=[
            pltpu.VMEM((_ROWS_W, 128), jnp.int32),
            pltpu.VMEM((_ROWS_W, 128), jnp.int32),
            pltpu.VMEM((_ROWS_W, 128), jnp.float32),
            pltpu.VMEM((_ROWS_W, 128), jnp.int32),
            pltpu.VMEM((4096,), jnp.float32),
            pltpu.VMEM_SHARED((_AWORDS,), jnp.float32),
            pltpu.SemaphoreType.DMA,
        ],
    )
    return run(src, dst, attr)


def _sig(z):
    return 1.0 / (1.0 + jnp.exp(-z))


def _rank_desc(score, n):
    si = score[:, :, None]
    sj = score[:, None, :]
    jj = lax.broadcasted_iota(jnp.int32, (_B, n, n), 1)
    ii = lax.broadcasted_iota(jnp.int32, (_B, n, n), 2)
    beats = (si > sj) | ((si == sj) & (jj < ii))
    return jnp.sum(beats.astype(jnp.int32), axis=1)


def _bmm(a, b):
    return lax.dot_general(a, b, (((2,), (1,)), ((0,), (0,))),
                           precision=lax.Precision.HIGHEST,
                           preferred_element_type=jnp.float32)


def _tc_body(a2, xr, n1w1, w2r1, b2r1, root1, bias1, p1w,
             n2w1, w2r2, b2r2, root2, bias2, p2w,
             m1W, m1b, bn1g, bn1b, m2W, m2b, bn2g, bn2b, smxW, smxb,
             out_ref, sc1_ref, sc2_ref):
    A = a2[0] + a2[1]
    xb = xr[...]

    h1w = jnp.maximum(n1w1[...], 0.0)
    W1 = b2r1[...][None]
    W1 = jnp.broadcast_to(W1, (_R, _R, _D1))
    for c in range(_NCLUST):
        W1 = W1 + h1w[:, c:c + 1][..., None] * w2r1[c][None]
    xwt = lax.dot_general(xb, W1, (((2,), (1,)), ((1,), (0,))),
                          precision=lax.Precision.HIGHEST,
                          preferred_element_type=jnp.float32)
    agg = lax.dot_general(A, xwt, (((2,), (0,)), ((0,), (1,))),
                          precision=lax.Precision.HIGHEST,
                          preferred_element_type=jnp.float32)
    rootp = lax.dot_general(xb, root1[...], (((2,), (0,)), ((), ())),
                            precision=lax.Precision.HIGHEST,
                            preferred_element_type=jnp.float32)
    h1 = agg + rootp + bias1[...][None]

    w1 = p1w[...]
    proj1 = jnp.sum(h1 * (w1 / (jnp.sqrt(jnp.sum(w1 * w1)) + 1e-16))[None],
                    axis=-1)
    score1n = _sig(proj1)
    rank1 = _rank_desc(score1n, _R)
    kk1 = lax.broadcasted_iota(jnp.int32, (_B, _K1, _R), 1)
    S1 = (kk1 == rank1[:, None, :]).astype(jnp.float32)
    nn1 = lax.broadcasted_iota(jnp.int32, (_B, _R, _K1), 2)
    S1T = (nn1 == rank1[:, :, None]).astype(jnp.float32)

    sh1 = h1 * score1n[:, :, None]
    x1 = _bmm(S1, sh1)
    vals1 = jnp.sum(S1 * score1n[:, None, :], axis=-1)
    sc1_ref[...] = _sig(vals1)

    A1 = _bmm(_bmm(S1, A), S1T)
    ii = lax.broadcasted_iota(jnp.int32, (_K1, _K1), 0)
    jj = lax.broadcasted_iota(jnp.int32, (_K1, _K1), 1)
    eye = (ii == jj).astype(jnp.float32)
    As = A1 + eye[None]
    A1a = _bmm(As, As) * (1.0 - eye)[None]

    h2w = jnp.maximum(n2w1[...], 0.0)
    W2 = jnp.broadcast_to(b2r2[...][None], (_R, _D1, _D2))
    for c in range(_NCLUST):
        W2 = W2 + h2w[:, c:c + 1][..., None] * w2r2[c][None]
    yt = lax.dot_general(sh1, W2, (((2,), (1,)), ((1,), (0,))),
                         precision=lax.Precision.HIGHEST,
                         preferred_element_type=jnp.float32)
    xw2 = lax.dot_general(S1, yt, (((2,), (0,)), ((0,), (1,))),
                          precision=lax.Precision.HIGHEST,
                          preferred_element_type=jnp.float32)
    root2p = lax.dot_general(x1, root2[...], (((2,), (0,)), ((), ())),
                             precision=lax.Precision.HIGHEST,
                             preferred_element_type=jnp.float32)
    h2 = _bmm(A1a, xw2) + root2p + bias2[...][None]

    w2 = p2w[...]
    proj2 = jnp.sum(h2 * (w2 / (jnp.sqrt(jnp.sum(w2 * w2)) + 1e-16))[None],
                    axis=-1)
    score2n = _sig(proj2)
    rank2 = _rank_desc(score2n, _K1)
    kk2 = lax.broadcasted_iota(jnp.int32, (_B, _K2, _K1), 1)
    S2 = (kk2 == rank2[:, None, :]).astype(jnp.float32)
    sh2 = h2 * score2n[:, :, None]
    x2 = _bmm(S2, sh2)
    vals2 = jnp.sum(S2 * score2n[:, None, :], axis=-1)
    sc2_ref[...] = _sig(vals2)

    read = jnp.concatenate([
        jnp.max(x1, axis=1), jnp.sum(x1, axis=1) / _K1,
        jnp.max(x2, axis=1), jnp.sum(x2, axis=1) / _K2,
    ], axis=-1)
    bnc = jnp.sqrt(jnp.float32(1.0 + 1e-5))
    z = jnp.dot(read, m1W[...], precision=lax.Precision.HIGHEST,
                preferred_element_type=jnp.float32) + m1b[...]
    z = jnp.where(z >= 0, z, 0.01 * z)
    z = bn1g[...] * z / bnc + bn1b[...]
    z = jnp.dot(z, m2W[...], precision=lax.Precision.HIGHEST,
                preferred_element_type=jnp.float32) + m2b[...]
    z = jnp.where(z >= 0, z, 0.01 * z)
    z = bn2g[...] * z / bnc + bn2b[...]
    logits = jnp.dot(z, smxW[...], precision=lax.Precision.HIGHEST,
                     preferred_element_type=jnp.float32) + smxb[...]
    mx = jnp.max(logits, axis=-1, keepdims=True)
    sh = logits - mx
    out_ref[...] = sh - jnp.log(jnp.sum(jnp.exp(sh), axis=-1, keepdims=True))


def _tc_forward(a2, xb, *weights, interpret=False):
    return pl.pallas_call(
        _tc_body,
        out_shape=[
            jax.ShapeDtypeStruct((_B, 2), jnp.float32),
            jax.ShapeDtypeStruct((_B, _K1), jnp.float32),
            jax.ShapeDtypeStruct((_B, _K2), jnp.float32),
        ],
        interpret=interpret,
    )(a2, xb, *weights)


def kernel(x, edge_index, edge_attr, pos, batch, n1_w1, n1_w2, n1_b2,
           conv1_root, conv1_bias, pool1_w, n2_w1, n2_w2, n2_b2, conv2_root,
           conv2_bias, pool2_w, mlp1_W, mlp1_b, bn1_g, bn1_b, mlp2_W, mlp2_b,
           bn2_g, bn2_b, smx_W, smx_b):
    src = edge_index[0].reshape(_EROWS, 128)
    dst = edge_index[1].reshape(_EROWS, 128)
    attr = edge_attr.reshape(_EROWS, 128)
    a2 = _build_adj(src, dst, attr).reshape(2, _B, _R, _R)

    out, sc1, sc2 = _tc_forward(
        a2, x.reshape(_B, _R, _R),
        n1_w1, n1_w2.reshape(_NCLUST, _R, _D1), n1_b2.reshape(_R, _D1),
        conv1_root, conv1_bias.reshape(1, _D1), pool1_w.reshape(1, _D1),
        n2_w1, n2_w2.reshape(_NCLUST, _D1, _D2), n2_b2.reshape(_D1, _D2),
        conv2_root, conv2_bias.reshape(1, _D2), pool2_w.reshape(1, _D2),
        mlp1_W, mlp1_b.reshape(1, -1), bn1_g.reshape(1, -1),
        bn1_b.reshape(1, -1), mlp2_W, mlp2_b.reshape(1, -1),
        bn2_g.reshape(1, -1), bn2_b.reshape(1, -1), smx_W,
        smx_b.reshape(1, -1))
    return (out, pool1_w, pool2_w, sc1, sc2)
